# Initial kernel scaffold; baseline (speedup 1.0000x reference)
import functools

import jax
import jax.numpy as jnp
from jax import lax
from jax.experimental import pallas as pl
from jax.experimental.pallas import tpu as pltpu
from jax.experimental.pallas import tpu_sc as plsc

B = 4096 * 50
D = 64
NW = 32
BPW = B // NW
C = 800
NCHUNK = BPW // C

_mesh = plsc.VectorSubcoreMesh(core_axis_name="c", subcore_axis_name="s")


@functools.partial(
    pl.kernel,
    mesh=_mesh,
    out_type=jax.ShapeDtypeStruct((B, D), jnp.float32),
    scratch_types=[
        pltpu.VMEM((2, C), jnp.int32),
        pltpu.VMEM((2, C, D), jnp.float32),
        pltpu.SemaphoreType.DMA,
    ],
)
def _gather_kernel(idx_hbm, table_hbm, out_hbm, idx_v, rows_v, gsem):
    wid = lax.axis_index("s") * 2 + lax.axis_index("c")
    base = wid * BPW

    pltpu.sync_copy(idx_hbm.at[pl.ds(base, C)], idx_v.at[0])
    prev = pltpu.async_copy(table_hbm.at[idx_v.at[0]], rows_v.at[0], gsem)
    prev_buf = 0

    for i in range(1, NCHUNK):
        buf = i & 1
        pltpu.sync_copy(idx_hbm.at[pl.ds(base + i * C, C)], idx_v.at[buf])
        cur = pltpu.async_copy(table_hbm.at[idx_v.at[buf]], rows_v.at[buf], gsem)
        prev.wait()
        pltpu.sync_copy(
            rows_v.at[prev_buf], out_hbm.at[pl.ds(base + (i - 1) * C, C)]
        )
        prev, prev_buf = cur, buf

    prev.wait()
    pltpu.sync_copy(
        rows_v.at[prev_buf], out_hbm.at[pl.ds(base + (NCHUNK - 1) * C, C)]
    )


def kernel(items, table):
    idx = items.reshape(-1).astype(jnp.int32)
    out = _gather_kernel(idx, table)
    return out.reshape(items.shape + (table.shape[1],))


# speedup vs baseline: 4.6484x; 4.6484x over previous
import functools

import jax
import jax.numpy as jnp
from jax import lax
from jax.experimental import pallas as pl
from jax.experimental.pallas import tpu as pltpu
from jax.experimental.pallas import tpu_sc as plsc

B = 4096 * 50
D = 64
NW = 32
BPW = B // NW
C = 800
NCHUNK = BPW // C

_mesh = plsc.VectorSubcoreMesh(core_axis_name="c", subcore_axis_name="s")


@functools.partial(
    pl.kernel,
    mesh=_mesh,
    out_type=jax.ShapeDtypeStruct((B, D), jnp.float32),
    compiler_params=pltpu.CompilerParams(use_tc_tiling_on_sc=False),
    scratch_types=[
        pltpu.VMEM((C,), jnp.int32),
        pltpu.VMEM((C,), jnp.int32),
        pltpu.VMEM((C, D), jnp.float32),
        pltpu.VMEM((C, D), jnp.float32),
        pltpu.SemaphoreType.DMA,
    ],
)
def _gather_kernel(idx_hbm, table_hbm, out_hbm, idx_v0, idx_v1, rows_v0,
                   rows_v1, gsem):
    wid = lax.axis_index("s") * 2 + lax.axis_index("c")
    base = wid * BPW
    idx_bufs = (idx_v0, idx_v1)
    row_bufs = (rows_v0, rows_v1)

    pltpu.sync_copy(idx_hbm.at[pl.ds(base, C)], idx_v0)
    prev = pltpu.async_copy(table_hbm.at[idx_v0], rows_v0, gsem)
    prev_buf = 0

    for i in range(1, NCHUNK):
        buf = i & 1
        pltpu.sync_copy(idx_hbm.at[pl.ds(base + i * C, C)], idx_bufs[buf])
        cur = pltpu.async_copy(table_hbm.at[idx_bufs[buf]], row_bufs[buf], gsem)
        prev.wait()
        pltpu.sync_copy(
            row_bufs[prev_buf], out_hbm.at[pl.ds(base + (i - 1) * C, C)]
        )
        prev, prev_buf = cur, buf

    prev.wait()
    pltpu.sync_copy(
        row_bufs[prev_buf], out_hbm.at[pl.ds(base + (NCHUNK - 1) * C, C)]
    )


def kernel(items, table):
    idx = items.reshape(-1).astype(jnp.int32)
    out = _gather_kernel(idx, table)
    return out.reshape(items.shape + (table.shape[1],))
